# Initial kernel scaffold; baseline (speedup 1.0000x reference)
import jax
import jax.numpy as jnp
from jax.experimental import pallas as pl


def kernel(x, edge_index, batch, num_nodes, y, con, eps, params):
    raise NotImplementedError("write your pallas kernel here")


# speedup vs baseline: 3.6371x; 3.6371x over previous
import functools

import jax
import jax.numpy as jnp
from jax import lax
from jax.experimental import pallas as pl
from jax.experimental.pallas import tpu as pltpu
from jax.experimental.pallas import tpu_sc as plsc

NN = 16384
BB = 64
LL = 256
EE = 262144
HDD = 376
HDP = 384
FC = 128
NSC = 2
NTI = 16
EB = 128
NH = NN // NSC
NPH = NH // NTI

_mesh = lambda: plsc.VectorSubcoreMesh(core_axis_name="c", subcore_axis_name="s")


def _agg_body(C, src_hbm, dst_hbm, *rest):
    hs = rest[:C]
    out_hbm = rest[C]
    sidx, didx, gbuf, acc = rest[C + 1:]
    cid = lax.axis_index("c")
    sid = lax.axis_index("s")
    ept = EE // NTI
    goff = cid * NH

    def run_chunk(c):
        def ib(j, _):
            nb = sid * NPH + j * EB
            pltpu.sync_copy(hs[c].at[pl.ds(goff + nb, EB)], gbuf)
            pltpu.sync_copy(gbuf, acc.at[pl.ds(nb, EB)])
            return 0
        lax.fori_loop(0, NPH // EB, ib, 0)
        plsc.subcore_barrier()

        def ebody(k, _):
            base = sid * ept + k * EB
            pltpu.sync_copy(src_hbm.at[pl.ds(base, EB)], sidx)
            pltpu.sync_copy(dst_hbm.at[pl.ds(base, EB)], didx)
            for v in range(EB // 16):
                d = didx[pl.ds(v * 16, 16)] - goff
                ok = (d >= 0) & (d < NH)
                didx[pl.ds(v * 16, 16)] = jnp.where(ok, d, NH)
            pltpu.sync_copy(hs[c].at[sidx], gbuf)
            pltpu.sync_copy(gbuf, acc.at[didx], add=True)
            return 0
        lax.fori_loop(0, ept // EB, ebody, 0)
        plsc.subcore_barrier()

        def ob(j, _):
            nb = sid * NPH + j * EB
            pltpu.sync_copy(acc.at[pl.ds(nb, EB)], gbuf)
            pltpu.sync_copy(gbuf, out_hbm.at[pl.ds(c * NN + goff + nb, EB)])
            return 0
        lax.fori_loop(0, NPH // EB, ob, 0)
        plsc.subcore_barrier()

    for c in range(C):
        run_chunk(c)


def _agg(C, src, dst, hchunks):
    k = pl.kernel(
        functools.partial(_agg_body, C),
        out_type=jax.ShapeDtypeStruct((C * NN, FC), jnp.float32),
        mesh=_mesh(),
        scratch_types=[
            pltpu.VMEM((EB,), jnp.int32),
            pltpu.VMEM((EB,), jnp.int32),
            pltpu.VMEM((EB, FC), jnp.float32),
            pltpu.VMEM_SHARED((NH + 8, FC), jnp.float32),
        ],
    )
    return k(src, dst, *hchunks)


_RB = 2048
_NRB = NN // _RB


def _dinv_body(db, o):
    deg = db[:, :16]
    o[...] = jnp.where(deg > 0, 1.0 / jnp.sqrt(deg), 0.0)


def _dinv(degfull):
    return pl.pallas_call(
        _dinv_body,
        grid=(_NRB,),
        in_specs=[pl.BlockSpec((_RB, FC), lambda i: (i, 0))],
        out_specs=pl.BlockSpec((_RB, 16), lambda i: (i, 0)),
        out_shape=jax.ShapeDtypeStruct((NN, 16), jnp.float32),
    )(degfull)


def _mm_body(xb, wb, db, ob):
    h = jnp.dot(xb[...], wb[...], preferred_element_type=jnp.float32)
    ob[...] = h * db[:, :1]


def _mm(C, fin, x, wp, dinv):
    fout = C * FC
    return pl.pallas_call(
        _mm_body,
        grid=(_NRB,),
        in_specs=[
            pl.BlockSpec((_RB, fin), lambda i: (i, 0)),
            pl.BlockSpec((fin, fout), lambda i: (0, 0)),
            pl.BlockSpec((_RB, 16), lambda i: (i, 0)),
        ],
        out_specs=pl.BlockSpec((_RB, fout), lambda i: (i, 0)),
        out_shape=jax.ShapeDtypeStruct((NN, fout), jnp.float32),
    )(x, wp, dinv)


def _epi_body(C, *refs):
    aggs = refs[:C]
    db, bb, ob = refs[C:]
    acat = jnp.concatenate([a[...] for a in aggs], axis=1)
    ob[...] = jnp.maximum(acat * db[:, :1] + bb[...], 0.0)


def _epi(C, aggchunks, dinv, bias):
    fout = C * FC
    return pl.pallas_call(
        functools.partial(_epi_body, C),
        grid=(_NRB,),
        in_specs=[pl.BlockSpec((_RB, FC), lambda i: (i, 0))] * C + [
            pl.BlockSpec((_RB, 16), lambda i: (i, 0)),
            pl.BlockSpec((1, fout), lambda i: (0, 0)),
        ],
        out_specs=pl.BlockSpec((_RB, fout), lambda i: (i, 0)),
        out_shape=jax.ShapeDtypeStruct((NN, fout), jnp.float32),
    )(*aggchunks, dinv, bias.reshape(1, fout))


def _cond_body(cb, wb, bb, yb, ob):
    ob[...] = (
        jnp.dot(cb[...], wb[...], preferred_element_type=jnp.float32)
        + bb[...] + yb[...]
    )


def _cond(con, condw, condb, y):
    cd = con.shape[1]
    return pl.pallas_call(
        _cond_body,
        in_specs=[
            pl.BlockSpec((BB, cd), lambda: (0, 0)),
            pl.BlockSpec((cd, HDD), lambda: (0, 0)),
            pl.BlockSpec((1, HDD), lambda: (0, 0)),
            pl.BlockSpec((BB, 1), lambda: (0, 0)),
        ],
        out_specs=pl.BlockSpec((BB, HDD), lambda: (0, 0)),
        out_shape=jax.ShapeDtypeStruct((BB, HDD), jnp.float32),
    )(con, condw, condb.reshape(1, HDD), y.reshape(BB, 1))


def _trans_body(xb, cb, ppb, dsb, mb):
    dsb[...] = (xb[...] + ppb[...]).reshape(LL, 1, 1, HDD)
    mb[...] = cb[...] == -999.0


def _trans(xr, col0, pp):
    d4, m3 = pl.pallas_call(
        _trans_body,
        grid=(BB,),
        in_specs=[
            pl.BlockSpec((LL, HDD), lambda b: (b, 0)),
            pl.BlockSpec((1, 1, LL), lambda b: (b, 0, 0)),
            pl.BlockSpec((1, HDD), lambda b: (0, 0)),
        ],
        out_specs=[
            pl.BlockSpec((LL, 1, 1, HDD), lambda b: (0, b, 0, 0)),
            pl.BlockSpec((1, 1, LL), lambda b: (b, 0, 0)),
        ],
        out_shape=[
            jax.ShapeDtypeStruct((LL, BB, 1, HDD), jnp.float32),
            jax.ShapeDtypeStruct((BB, 1, LL), jnp.bool_),
        ],
    )(xr, col0.reshape(BB, 1, LL), pp.reshape(1, HDD))
    return d4.reshape(LL, BB, HDD), m3.reshape(BB, LL)


_LB = 32
_NLB = LL // _LB


def _vae_body(dsb, epsb, cab, m1w, m1b, m2w, m2b, v1w, v1b, v2w, v2b,
              zb, klb, accr):
    ds2 = dsb[...].reshape(_LB * BB, HDD)
    h1 = jnp.maximum(
        jnp.dot(ds2, m1w[...], preferred_element_type=jnp.float32) + m1b[...], 0.0)
    mu = jnp.dot(h1, m2w[...], preferred_element_type=jnp.float32) + m2b[...]
    g1 = jnp.maximum(
        jnp.dot(ds2, v1w[...], preferred_element_type=jnp.float32) + v1b[...], 0.0)
    lv = jnp.dot(g1, v2w[...], preferred_element_type=jnp.float32) + v2b[...]
    zlv = -jnp.abs(lv)

    li = pl.program_id(0)

    @pl.when(li == 0)
    def _():
        accr[0] = 0.0

    accr[0] += jnp.sum(1.0 + zlv - mu * mu - jnp.exp(zlv))

    @pl.when(li == _NLB - 1)
    def _():
        klb[...] = (accr[0] * (-0.5 / 64.0)).reshape(1, 1)

    z3 = (mu.reshape(_LB, BB, HDD)
          + jnp.exp(zlv * 0.5).reshape(_LB, BB, HDD) * epsb[...]
          + cab[...])
    zb[...] = z3


def _vae(d_seq, eps, ca, p):
    return pl.pallas_call(
        _vae_body,
        grid=(_NLB,),
        in_specs=[
            pl.BlockSpec((_LB, BB, HDD), lambda l: (l, 0, 0)),
            pl.BlockSpec((_LB, BB, HDD), lambda l: (l, 0, 0)),
            pl.BlockSpec((1, BB, HDD), lambda l: (0, 0, 0)),
            pl.BlockSpec((HDD, HDD), lambda l: (0, 0)),
            pl.BlockSpec((1, HDD), lambda l: (0, 0)),
            pl.BlockSpec((HDD, HDD), lambda l: (0, 0)),
            pl.BlockSpec((1, HDD), lambda l: (0, 0)),
            pl.BlockSpec((HDD, HDD), lambda l: (0, 0)),
            pl.BlockSpec((1, HDD), lambda l: (0, 0)),
            pl.BlockSpec((HDD, HDD), lambda l: (0, 0)),
            pl.BlockSpec((1, HDD), lambda l: (0, 0)),
        ],
        out_specs=[
            pl.BlockSpec((_LB, BB, HDD), lambda l: (l, 0, 0)),
            pl.BlockSpec((1, 1), lambda l: (0, 0)),
        ],
        out_shape=[
            jax.ShapeDtypeStruct((LL, BB, HDD), jnp.float32),
            jax.ShapeDtypeStruct((1, 1), jnp.float32),
        ],
        scratch_shapes=[pltpu.SMEM((1,), jnp.float32)],
    )(d_seq, eps, ca.reshape(1, BB, HDD),
      p['m1W'], p['m1b'].reshape(1, HDD), p['m2W'], p['m2b'].reshape(1, HDD),
      p['v1W'], p['v1b'].reshape(1, HDD), p['v2W'], p['v2b'].reshape(1, HDD))


def _segmax_body(xb, ob):
    ob[...] = jnp.max(xb[...], axis=1, keepdims=True)


def _segmax(xr3):
    out = pl.pallas_call(
        _segmax_body,
        grid=(BB,),
        in_specs=[pl.BlockSpec((1, LL, HDP), lambda b: (b, 0, 0))],
        out_specs=pl.BlockSpec((1, 1, HDP), lambda b: (b, 0, 0)),
        out_shape=jax.ShapeDtypeStruct((BB, 1, HDP), jnp.float32),
    )(xr3)
    return out.reshape(BB, HDP)


def _pmvo_body(xb, w1, b1, w2, b2, ob):
    h = jnp.maximum(
        jnp.dot(xb[...], w1[...], preferred_element_type=jnp.float32) + b1[...], 0.0)
    ob[...] = jnp.dot(h, w2[...], preferred_element_type=jnp.float32) + b2[...]


def _pmvo(x2, f1wp, f1b, f2w, f2b):
    return pl.pallas_call(
        _pmvo_body,
        in_specs=[
            pl.BlockSpec((BB, HDP), lambda: (0, 0)),
            pl.BlockSpec((HDP, 1024), lambda: (0, 0)),
            pl.BlockSpec((1, 1024), lambda: (0, 0)),
            pl.BlockSpec((1024, 128), lambda: (0, 0)),
            pl.BlockSpec((1, 128), lambda: (0, 0)),
        ],
        out_specs=pl.BlockSpec((BB, 128), lambda: (0, 0)),
        out_shape=jax.ShapeDtypeStruct((BB, 128), jnp.float32),
    )(x2, f1wp, f1b.reshape(1, 1024), f2w, f2b.reshape(1, 128))


def _pad2(a, r, c):
    return jnp.pad(a, ((0, r - a.shape[0]), (0, c - a.shape[1])))


def kernel(x, edge_index, batch, num_nodes, y, con, eps, params):
    p = params
    src = edge_index[0]
    dst = edge_index[1]

    ones = jnp.ones((NN, FC), jnp.float32)
    degfull = _agg(1, src, dst, [ones])
    dinv = _dinv(degfull)

    def layer(xin, C, fin, W, b):
        wp = _pad2(W, fin, C * FC)
        bp = jnp.pad(b, (0, C * FC - b.shape[0]))
        h2 = _mm(C, fin, xin, wp, dinv)
        hchunks = [h2[:, c * FC:(c + 1) * FC] for c in range(C)]
        agg = _agg(C, src, dst, hchunks)
        aggchunks = [agg[c * NN:(c + 1) * NN] for c in range(C)]
        return _epi(C, aggchunks, dinv, bp)

    xp = jnp.pad(x, ((0, 0), (0, 2)))
    x1 = layer(xp, 2, 96, p['W1'], p['b1'])
    x2in = layer(x1, 3, 2 * FC, p['W2'], p['b2'])
    xr = layer(x2in, 3, 3 * FC, p['W3'], p['b3'])

    ca = _cond(con, p['condW'], p['condb'], y)
    col0 = xr[:, 0].reshape(BB, LL)
    d_seq, mask = _trans(xr[:, :HDD], col0, p['pp'])
    z, kl2 = _vae(d_seq, eps, ca, p)

    x2 = _segmax(xr.reshape(BB, LL, HDP))
    f1wp = jnp.pad(p['f1W'], ((0, HDP - HDD), (0, 0)))
    pmvo = _pmvo(x2, f1wp, p['f1b'], p['f2W'], p['f2b'])

    return d_seq, z, mask, pmvo, kl2[0, 0]


# speedup vs baseline: 5.4328x; 1.4937x over previous
import functools

import jax
import jax.numpy as jnp
from jax import lax
from jax.experimental import pallas as pl
from jax.experimental.pallas import tpu as pltpu
from jax.experimental.pallas import tpu_sc as plsc

NN = 16384
BB = 64
LL = 256
EE = 262144
HDD = 376
HDP = 384
FC = 128
NSC = 2
NTI = 16
EB = 128
NH = NN // NSC
NPH = NH // NTI

_mesh = lambda: plsc.VectorSubcoreMesh(core_axis_name="c", subcore_axis_name="s")


NBLK = (EE // NTI) // EB
NBUF = 2


def _agg_body(C, src_hbm, dst3_hbm, *rest):
    hs = rest[:C]
    out_hbm = rest[C]
    sc = rest[C + 1:]
    didx3 = sc[0]
    sring = sc[1]
    gbufs = sc[2:2 + NBUF]
    sgs = sc[2 + NBUF:2 + 2 * NBUF]
    sss = sc[2 + 2 * NBUF:2 + 3 * NBUF]
    sis = sc[2 + 3 * NBUF:2 + 4 * NBUF]
    acc = sc[-1]
    cid = lax.axis_index("c")
    sid = lax.axis_index("s")
    ept = EE // NTI
    goff = cid * NH

    pltpu.sync_copy(dst3_hbm.at[pl.ds(sid * NBLK, NBLK)], didx3)

    def rm(j, _):
        def rv(v, _2):
            d = didx3[j, pl.ds(v * 16, 16)] - goff
            ok = (d >= 0) & (d < NH)
            didx3[j, pl.ds(v * 16, 16)] = jnp.where(ok, d, NH)
            return 0
        lax.fori_loop(0, EB // 16, rv, 0)
        return 0
    lax.fori_loop(0, NBLK, rm, 0)

    def idx(j, b):
        return src_hbm.at[pl.ds(sid * ept + j * EB, EB)], sring.at[b], sis[b]

    def gat(c, j, b):
        return hs[c].at[sring.at[b]], gbufs[b], sgs[b]

    def sca(c, j, b):
        return gbufs[b], acc.at[didx3.at[j]], sss[b]

    def run_chunk(c):
        def ib(j, _):
            nb = sid * NPH + j * EB
            pltpu.sync_copy(hs[c].at[pl.ds(goff + nb, EB)], gbufs[0])
            pltpu.sync_copy(gbufs[0], acc.at[pl.ds(nb, EB)])
            return 0
        lax.fori_loop(0, NPH // EB, ib, 0)
        plsc.subcore_barrier()

        for b in range(NBUF):
            s, d, m = idx(b, b)
            pltpu.async_copy(s, d, m)
        for b in range(NBUF):
            s, d, m = idx(b, b)
            pltpu.make_async_copy(s, d, m).wait()
            s, d, m = gat(c, b, b)
            pltpu.async_copy(s, d, m)

        def rounds(jp, _):
            j0 = jp * NBUF
            for b in range(NBUF):
                j = j0 + b
                s, d, m = gat(c, j, b)
                pltpu.make_async_copy(s, d, m).wait()
                s, d, m = sca(c, j, b)
                pltpu.async_copy(s, d, m, add=True)
                jn = j + NBUF

                @pl.when(jn < NBLK)
                def _():
                    s2, d2, m2 = idx(jn, b)
                    pltpu.async_copy(s2, d2, m2)
            for b in range(NBUF):
                j = j0 + b
                jn = j + NBUF

                @pl.when(jn < NBLK)
                def _():
                    s2, d2, m2 = sca(c, j, b)
                    pltpu.make_async_copy(s2, d2, m2).wait()
                    s2, d2, m2 = idx(jn, b)
                    pltpu.make_async_copy(s2, d2, m2).wait()
                    s2, d2, m2 = gat(c, jn, b)
                    pltpu.async_copy(s2, d2, m2)
            return 0
        lax.fori_loop(0, NBLK // NBUF, rounds, 0)

        for b in range(NBUF):
            s, d, m = sca(c, NBLK - NBUF + b, b)
            pltpu.make_async_copy(s, d, m).wait()
        plsc.subcore_barrier()

        def ob(j, _):
            nb = sid * NPH + j * EB
            pltpu.sync_copy(acc.at[pl.ds(nb, EB)], gbufs[0])
            pltpu.sync_copy(gbufs[0], out_hbm.at[pl.ds(c * NN + goff + nb, EB)])
            return 0
        lax.fori_loop(0, NPH // EB, ob, 0)
        plsc.subcore_barrier()

    for c in range(C):
        run_chunk(c)


def _agg(C, src, dst, hchunks):
    k = pl.kernel(
        functools.partial(_agg_body, C),
        out_type=jax.ShapeDtypeStruct((C * NN, FC), jnp.float32),
        mesh=_mesh(),
        scratch_types=(
            [
                pltpu.VMEM((NBLK, EB), jnp.int32),
                pltpu.VMEM((NBUF, EB), jnp.int32),
            ]
            + [pltpu.VMEM((EB, FC), jnp.float32)] * NBUF
            + [pltpu.SemaphoreType.DMA] * (3 * NBUF)
            + [pltpu.VMEM_SHARED((NH + 8, FC), jnp.float32)]
        ),
    )
    return k(src, dst.reshape(EE // EB, EB), *hchunks)


_RB = 2048
_NRB = NN // _RB


def _dinv_body(db, o):
    deg = db[:, :16]
    o[...] = jnp.where(deg > 0, 1.0 / jnp.sqrt(deg), 0.0)


def _dinv(degfull):
    return pl.pallas_call(
        _dinv_body,
        grid=(_NRB,),
        in_specs=[pl.BlockSpec((_RB, FC), lambda i: (i, 0))],
        out_specs=pl.BlockSpec((_RB, 16), lambda i: (i, 0)),
        out_shape=jax.ShapeDtypeStruct((NN, 16), jnp.float32),
    )(degfull)


def _mm_body(xb, wb, db, ob):
    h = jnp.dot(xb[...], wb[...], preferred_element_type=jnp.float32)
    ob[...] = h * db[:, :1]


def _mm(C, fin, x, wp, dinv):
    fout = C * FC
    return pl.pallas_call(
        _mm_body,
        grid=(_NRB,),
        in_specs=[
            pl.BlockSpec((_RB, fin), lambda i: (i, 0)),
            pl.BlockSpec((fin, fout), lambda i: (0, 0)),
            pl.BlockSpec((_RB, 16), lambda i: (i, 0)),
        ],
        out_specs=pl.BlockSpec((_RB, fout), lambda i: (i, 0)),
        out_shape=jax.ShapeDtypeStruct((NN, fout), jnp.float32),
    )(x, wp, dinv)


def _epi_body(C, *refs):
    aggs = refs[:C]
    db, bb, ob = refs[C:]
    acat = jnp.concatenate([a[...] for a in aggs], axis=1)
    ob[...] = jnp.maximum(acat * db[:, :1] + bb[...], 0.0)


def _epi(C, aggchunks, dinv, bias):
    fout = C * FC
    return pl.pallas_call(
        functools.partial(_epi_body, C),
        grid=(_NRB,),
        in_specs=[pl.BlockSpec((_RB, FC), lambda i: (i, 0))] * C + [
            pl.BlockSpec((_RB, 16), lambda i: (i, 0)),
            pl.BlockSpec((1, fout), lambda i: (0, 0)),
        ],
        out_specs=pl.BlockSpec((_RB, fout), lambda i: (i, 0)),
        out_shape=jax.ShapeDtypeStruct((NN, fout), jnp.float32),
    )(*aggchunks, dinv, bias.reshape(1, fout))


def _cond_body(cb, wb, bb, yb, ob):
    ob[...] = (
        jnp.dot(cb[...], wb[...], preferred_element_type=jnp.float32)
        + bb[...] + yb[...]
    )


def _cond(con, condw, condb, y):
    cd = con.shape[1]
    return pl.pallas_call(
        _cond_body,
        in_specs=[
            pl.BlockSpec((BB, cd), lambda: (0, 0)),
            pl.BlockSpec((cd, HDD), lambda: (0, 0)),
            pl.BlockSpec((1, HDD), lambda: (0, 0)),
            pl.BlockSpec((BB, 1), lambda: (0, 0)),
        ],
        out_specs=pl.BlockSpec((BB, HDD), lambda: (0, 0)),
        out_shape=jax.ShapeDtypeStruct((BB, HDD), jnp.float32),
    )(con, condw, condb.reshape(1, HDD), y.reshape(BB, 1))


def _trans_body(xb, cb, ppb, dsb, mb):
    dsb[...] = (xb[...] + ppb[...]).reshape(LL, 1, 1, HDD)
    mb[...] = cb[...] == -999.0


def _trans(xr, col0, pp):
    d4, m3 = pl.pallas_call(
        _trans_body,
        grid=(BB,),
        in_specs=[
            pl.BlockSpec((LL, HDD), lambda b: (b, 0)),
            pl.BlockSpec((1, 1, LL), lambda b: (b, 0, 0)),
            pl.BlockSpec((1, HDD), lambda b: (0, 0)),
        ],
        out_specs=[
            pl.BlockSpec((LL, 1, 1, HDD), lambda b: (0, b, 0, 0)),
            pl.BlockSpec((1, 1, LL), lambda b: (b, 0, 0)),
        ],
        out_shape=[
            jax.ShapeDtypeStruct((LL, BB, 1, HDD), jnp.float32),
            jax.ShapeDtypeStruct((BB, 1, LL), jnp.bool_),
        ],
    )(xr, col0.reshape(BB, 1, LL), pp.reshape(1, HDD))
    return d4.reshape(LL, BB, HDD), m3.reshape(BB, LL)


_LB = 32
_NLB = LL // _LB


def _vae_body(dsb, epsb, cab, m1w, m1b, m2w, m2b, v1w, v1b, v2w, v2b,
              zb, klb, accr):
    ds2 = dsb[...].reshape(_LB * BB, HDD)
    h1 = jnp.maximum(
        jnp.dot(ds2, m1w[...], preferred_element_type=jnp.float32) + m1b[...], 0.0)
    mu = jnp.dot(h1, m2w[...], preferred_element_type=jnp.float32) + m2b[...]
    g1 = jnp.maximum(
        jnp.dot(ds2, v1w[...], preferred_element_type=jnp.float32) + v1b[...], 0.0)
    lv = jnp.dot(g1, v2w[...], preferred_element_type=jnp.float32) + v2b[...]
    zlv = -jnp.abs(lv)

    li = pl.program_id(0)

    @pl.when(li == 0)
    def _():
        accr[0] = 0.0

    accr[0] += jnp.sum(1.0 + zlv - mu * mu - jnp.exp(zlv))

    @pl.when(li == _NLB - 1)
    def _():
        klb[...] = (accr[0] * (-0.5 / 64.0)).reshape(1, 1)

    z3 = (mu.reshape(_LB, BB, HDD)
          + jnp.exp(zlv * 0.5).reshape(_LB, BB, HDD) * epsb[...]
          + cab[...])
    zb[...] = z3


def _vae(d_seq, eps, ca, p):
    return pl.pallas_call(
        _vae_body,
        grid=(_NLB,),
        in_specs=[
            pl.BlockSpec((_LB, BB, HDD), lambda l: (l, 0, 0)),
            pl.BlockSpec((_LB, BB, HDD), lambda l: (l, 0, 0)),
            pl.BlockSpec((1, BB, HDD), lambda l: (0, 0, 0)),
            pl.BlockSpec((HDD, HDD), lambda l: (0, 0)),
            pl.BlockSpec((1, HDD), lambda l: (0, 0)),
            pl.BlockSpec((HDD, HDD), lambda l: (0, 0)),
            pl.BlockSpec((1, HDD), lambda l: (0, 0)),
            pl.BlockSpec((HDD, HDD), lambda l: (0, 0)),
            pl.BlockSpec((1, HDD), lambda l: (0, 0)),
            pl.BlockSpec((HDD, HDD), lambda l: (0, 0)),
            pl.BlockSpec((1, HDD), lambda l: (0, 0)),
        ],
        out_specs=[
            pl.BlockSpec((_LB, BB, HDD), lambda l: (l, 0, 0)),
            pl.BlockSpec((1, 1), lambda l: (0, 0)),
        ],
        out_shape=[
            jax.ShapeDtypeStruct((LL, BB, HDD), jnp.float32),
            jax.ShapeDtypeStruct((1, 1), jnp.float32),
        ],
        scratch_shapes=[pltpu.SMEM((1,), jnp.float32)],
    )(d_seq, eps, ca.reshape(1, BB, HDD),
      p['m1W'], p['m1b'].reshape(1, HDD), p['m2W'], p['m2b'].reshape(1, HDD),
      p['v1W'], p['v1b'].reshape(1, HDD), p['v2W'], p['v2b'].reshape(1, HDD))


def _segmax_body(xb, ob):
    ob[...] = jnp.max(xb[...], axis=1, keepdims=True)


def _segmax(xr3):
    out = pl.pallas_call(
        _segmax_body,
        grid=(BB,),
        in_specs=[pl.BlockSpec((1, LL, HDP), lambda b: (b, 0, 0))],
        out_specs=pl.BlockSpec((1, 1, HDP), lambda b: (b, 0, 0)),
        out_shape=jax.ShapeDtypeStruct((BB, 1, HDP), jnp.float32),
    )(xr3)
    return out.reshape(BB, HDP)


def _pmvo_body(xb, w1, b1, w2, b2, ob):
    h = jnp.maximum(
        jnp.dot(xb[...], w1[...], preferred_element_type=jnp.float32) + b1[...], 0.0)
    ob[...] = jnp.dot(h, w2[...], preferred_element_type=jnp.float32) + b2[...]


def _pmvo(x2, f1wp, f1b, f2w, f2b):
    return pl.pallas_call(
        _pmvo_body,
        in_specs=[
            pl.BlockSpec((BB, HDP), lambda: (0, 0)),
            pl.BlockSpec((HDP, 1024), lambda: (0, 0)),
            pl.BlockSpec((1, 1024), lambda: (0, 0)),
            pl.BlockSpec((1024, 128), lambda: (0, 0)),
            pl.BlockSpec((1, 128), lambda: (0, 0)),
        ],
        out_specs=pl.BlockSpec((BB, 128), lambda: (0, 0)),
        out_shape=jax.ShapeDtypeStruct((BB, 128), jnp.float32),
    )(x2, f1wp, f1b.reshape(1, 1024), f2w, f2b.reshape(1, 128))


def _pad2(a, r, c):
    return jnp.pad(a, ((0, r - a.shape[0]), (0, c - a.shape[1])))


def kernel(x, edge_index, batch, num_nodes, y, con, eps, params):
    p = params
    src = edge_index[0]
    dst = edge_index[1]

    ones = jnp.ones((NN, FC), jnp.float32)
    degfull = _agg(1, src, dst, [ones])
    dinv = _dinv(degfull)

    def layer(xin, C, fin, W, b):
        wp = _pad2(W, fin, C * FC)
        bp = jnp.pad(b, (0, C * FC - b.shape[0]))
        h2 = _mm(C, fin, xin, wp, dinv)
        hchunks = [h2[:, c * FC:(c + 1) * FC] for c in range(C)]
        agg = _agg(C, src, dst, hchunks)
        aggchunks = [agg[c * NN:(c + 1) * NN] for c in range(C)]
        return _epi(C, aggchunks, dinv, bp)

    xp = jnp.pad(x, ((0, 0), (0, 2)))
    x1 = layer(xp, 2, 96, p['W1'], p['b1'])
    x2in = layer(x1, 3, 2 * FC, p['W2'], p['b2'])
    xr = layer(x2in, 3, 3 * FC, p['W3'], p['b3'])

    ca = _cond(con, p['condW'], p['condb'], y)
    col0 = xr[:, 0].reshape(BB, LL)
    d_seq, mask = _trans(xr[:, :HDD], col0, p['pp'])
    z, kl2 = _vae(d_seq, eps, ca, p)

    x2 = _segmax(xr.reshape(BB, LL, HDP))
    f1wp = jnp.pad(p['f1W'], ((0, HDP - HDD), (0, 0)))
    pmvo = _pmvo(x2, f1wp, p['f1b'], p['f2W'], p['f2b'])

    return d_seq, z, mask, pmvo, kl2[0, 0]


# speedup vs baseline: 5.8594x; 1.0785x over previous
import functools

import jax
import jax.numpy as jnp
from jax import lax
from jax.experimental import pallas as pl
from jax.experimental.pallas import tpu as pltpu
from jax.experimental.pallas import tpu_sc as plsc

NN = 16384
BB = 64
LL = 256
EE = 262144
HDD = 376
HDP = 384
FC = 128
NSC = 2
NTI = 16
EB = 128
NH = NN // NSC
NPH = NH // NTI

_mesh = lambda: plsc.VectorSubcoreMesh(core_axis_name="c", subcore_axis_name="s")


NBLK = (EE // NTI) // EB
NBUF = 3


def _agg_body(C, src_hbm, dst_hbm, *rest):
    hs = rest[:C]
    out_hbm = rest[C]
    sc = rest[C + 1:]
    sring = sc[0]
    dring = sc[1]
    gbufs = sc[2:2 + NBUF]
    sgs = sc[2 + NBUF:2 + 2 * NBUF]
    sss = sc[2 + 2 * NBUF:2 + 3 * NBUF]
    sis = sc[2 + 3 * NBUF:2 + 4 * NBUF]
    sds = sc[2 + 4 * NBUF:2 + 5 * NBUF]
    acc = sc[-1]
    cid = lax.axis_index("c")
    sid = lax.axis_index("s")
    ept = EE // NTI
    goff = cid * NH

    def sidx(j, b):
        return src_hbm.at[pl.ds(sid * ept + j * EB, EB)], sring.at[b], sis[b]

    def didx(j, b):
        return dst_hbm.at[pl.ds(sid * ept + j * EB, EB)], dring.at[b], sds[b]

    def gat(c, j, b):
        return hs[c].at[sring.at[b]], gbufs[b], sgs[b]

    def sca(c, j, b):
        return gbufs[b], acc.at[dring.at[b]], sss[b]

    def remap(b):
        for v in range(EB // 16):
            d = dring[b, pl.ds(v * 16, 16)] - goff
            ok = (d >= 0) & (d < NH)
            dring[b, pl.ds(v * 16, 16)] = jnp.where(ok, d, NH)

    def fetch(j, b):
        for f in (sidx, didx):
            s, d, m = f(j, b)
            pltpu.async_copy(s, d, m)

    def open_block(c, j, b):
        for f in (sidx, didx):
            s, d, m = f(j, b)
            pltpu.make_async_copy(s, d, m).wait()
        remap(b)
        s, d, m = gat(c, j, b)
        pltpu.async_copy(s, d, m)

    def run_chunk(c):
        def ib(j, _):
            nb = sid * NPH + j * EB
            pltpu.sync_copy(hs[c].at[pl.ds(goff + nb, EB)], gbufs[0])
            pltpu.sync_copy(gbufs[0], acc.at[pl.ds(nb, EB)])
            return 0
        lax.fori_loop(0, NPH // EB, ib, 0)
        plsc.subcore_barrier()

        for b in range(NBUF):
            fetch(b, b)
        for b in range(NBUF):
            open_block(c, b, b)

        def rounds(jp, _):
            j0 = jp * NBUF
            for b in range(NBUF):
                j = j0 + b

                @pl.when(j < NBLK)
                def _():
                    s, d, m = gat(c, j, b)
                    pltpu.make_async_copy(s, d, m).wait()
                    s, d, m = sca(c, j, b)
                    pltpu.async_copy(s, d, m, add=True)
            for b in range(NBUF):
                jn = j0 + b + NBUF

                @pl.when(jn < NBLK)
                def _():
                    s2, d2, m2 = sca(c, j0 + b, b)
                    pltpu.make_async_copy(s2, d2, m2).wait()
                    fetch(jn, b)
                    open_block(c, jn, b)
            return 0
        lax.fori_loop(0, (NBLK + NBUF - 1) // NBUF, rounds, 0)

        for b in range(NBUF):
            s, d, m = sca(c, NBLK - NBUF + b, b)
            pltpu.make_async_copy(s, d, m).wait()
        plsc.subcore_barrier()

        def ob(j, _):
            nb = sid * NPH + j * EB
            pltpu.sync_copy(acc.at[pl.ds(nb, EB)], gbufs[0])
            pltpu.sync_copy(gbufs[0], out_hbm.at[pl.ds(c * NN + goff + nb, EB)])
            return 0
        lax.fori_loop(0, NPH // EB, ob, 0)
        plsc.subcore_barrier()

    for c in range(C):
        run_chunk(c)


def _agg(C, src, dst, hchunks):
    k = pl.kernel(
        functools.partial(_agg_body, C),
        out_type=jax.ShapeDtypeStruct((C * NN, FC), jnp.float32),
        mesh=_mesh(),
        scratch_types=(
            [
                pltpu.VMEM((NBUF, EB), jnp.int32),
                pltpu.VMEM((NBUF, EB), jnp.int32),
            ]
            + [pltpu.VMEM((EB, FC), jnp.float32)] * NBUF
            + [pltpu.SemaphoreType.DMA] * (4 * NBUF)
            + [pltpu.VMEM_SHARED((NH + 8, FC), jnp.float32)]
        ),
    )
    return k(src, dst, *hchunks)


_RB = 2048
_NRB = NN // _RB


def _dinv_body(db, o):
    deg = db[:, :16]
    o[...] = jnp.where(deg > 0, 1.0 / jnp.sqrt(deg), 0.0)


def _dinv(degfull):
    return pl.pallas_call(
        _dinv_body,
        grid=(_NRB,),
        in_specs=[pl.BlockSpec((_RB, FC), lambda i: (i, 0))],
        out_specs=pl.BlockSpec((_RB, 16), lambda i: (i, 0)),
        out_shape=jax.ShapeDtypeStruct((NN, 16), jnp.float32),
    )(degfull)


def _mm_body(xb, wb, db, ob):
    h = jnp.dot(xb[...], wb[...], preferred_element_type=jnp.float32)
    ob[...] = h * db[:, :1]


def _mm(C, fin, x, wp, dinv):
    fout = C * FC
    return pl.pallas_call(
        _mm_body,
        grid=(_NRB,),
        in_specs=[
            pl.BlockSpec((_RB, fin), lambda i: (i, 0)),
            pl.BlockSpec((fin, fout), lambda i: (0, 0)),
            pl.BlockSpec((_RB, 16), lambda i: (i, 0)),
        ],
        out_specs=pl.BlockSpec((_RB, fout), lambda i: (i, 0)),
        out_shape=jax.ShapeDtypeStruct((NN, fout), jnp.float32),
    )(x, wp, dinv)


def _epi_body(C, *refs):
    aggs = refs[:C]
    db, bb, ob = refs[C:]
    acat = jnp.concatenate([a[...] for a in aggs], axis=1)
    ob[...] = jnp.maximum(acat * db[:, :1] + bb[...], 0.0)


def _epi(C, aggchunks, dinv, bias):
    fout = C * FC
    return pl.pallas_call(
        functools.partial(_epi_body, C),
        grid=(_NRB,),
        in_specs=[pl.BlockSpec((_RB, FC), lambda i: (i, 0))] * C + [
            pl.BlockSpec((_RB, 16), lambda i: (i, 0)),
            pl.BlockSpec((1, fout), lambda i: (0, 0)),
        ],
        out_specs=pl.BlockSpec((_RB, fout), lambda i: (i, 0)),
        out_shape=jax.ShapeDtypeStruct((NN, fout), jnp.float32),
    )(*aggchunks, dinv, bias.reshape(1, fout))


def _cond_body(cb, wb, bb, yb, ob):
    ob[...] = (
        jnp.dot(cb[...], wb[...], preferred_element_type=jnp.float32)
        + bb[...] + yb[...]
    )


def _cond(con, condw, condb, y):
    cd = con.shape[1]
    return pl.pallas_call(
        _cond_body,
        in_specs=[
            pl.BlockSpec((BB, cd), lambda: (0, 0)),
            pl.BlockSpec((cd, HDD), lambda: (0, 0)),
            pl.BlockSpec((1, HDD), lambda: (0, 0)),
            pl.BlockSpec((BB, 1), lambda: (0, 0)),
        ],
        out_specs=pl.BlockSpec((BB, HDD), lambda: (0, 0)),
        out_shape=jax.ShapeDtypeStruct((BB, HDD), jnp.float32),
    )(con, condw, condb.reshape(1, HDD), y.reshape(BB, 1))


def _trans_body(xb, cb, ppb, dsb, mb):
    dsb[...] = (xb[...] + ppb[...]).reshape(LL, 1, 1, HDD)
    mb[...] = cb[...] == -999.0


def _trans(xr, col0, pp):
    d4, m3 = pl.pallas_call(
        _trans_body,
        grid=(BB,),
        in_specs=[
            pl.BlockSpec((LL, HDD), lambda b: (b, 0)),
            pl.BlockSpec((1, 1, LL), lambda b: (b, 0, 0)),
            pl.BlockSpec((1, HDD), lambda b: (0, 0)),
        ],
        out_specs=[
            pl.BlockSpec((LL, 1, 1, HDD), lambda b: (0, b, 0, 0)),
            pl.BlockSpec((1, 1, LL), lambda b: (b, 0, 0)),
        ],
        out_shape=[
            jax.ShapeDtypeStruct((LL, BB, 1, HDD), jnp.float32),
            jax.ShapeDtypeStruct((BB, 1, LL), jnp.bool_),
        ],
    )(xr, col0.reshape(BB, 1, LL), pp.reshape(1, HDD))
    return d4.reshape(LL, BB, HDD), m3.reshape(BB, LL)


_LB = 32
_NLB = LL // _LB


def _vae_body(dsb, epsb, cab, m1w, m1b, m2w, m2b, v1w, v1b, v2w, v2b,
              zb, klb, accr):
    ds2 = dsb[...].reshape(_LB * BB, HDD)
    h1 = jnp.maximum(
        jnp.dot(ds2, m1w[...], preferred_element_type=jnp.float32) + m1b[...], 0.0)
    mu = jnp.dot(h1, m2w[...], preferred_element_type=jnp.float32) + m2b[...]
    g1 = jnp.maximum(
        jnp.dot(ds2, v1w[...], preferred_element_type=jnp.float32) + v1b[...], 0.0)
    lv = jnp.dot(g1, v2w[...], preferred_element_type=jnp.float32) + v2b[...]
    zlv = -jnp.abs(lv)

    li = pl.program_id(0)

    @pl.when(li == 0)
    def _():
        accr[0] = 0.0

    accr[0] += jnp.sum(1.0 + zlv - mu * mu - jnp.exp(zlv))

    @pl.when(li == _NLB - 1)
    def _():
        klb[...] = (accr[0] * (-0.5 / 64.0)).reshape(1, 1)

    z3 = (mu.reshape(_LB, BB, HDD)
          + jnp.exp(zlv * 0.5).reshape(_LB, BB, HDD) * epsb[...]
          + cab[...])
    zb[...] = z3


def _vae(d_seq, eps, ca, p):
    return pl.pallas_call(
        _vae_body,
        grid=(_NLB,),
        in_specs=[
            pl.BlockSpec((_LB, BB, HDD), lambda l: (l, 0, 0)),
            pl.BlockSpec((_LB, BB, HDD), lambda l: (l, 0, 0)),
            pl.BlockSpec((1, BB, HDD), lambda l: (0, 0, 0)),
            pl.BlockSpec((HDD, HDD), lambda l: (0, 0)),
            pl.BlockSpec((1, HDD), lambda l: (0, 0)),
            pl.BlockSpec((HDD, HDD), lambda l: (0, 0)),
            pl.BlockSpec((1, HDD), lambda l: (0, 0)),
            pl.BlockSpec((HDD, HDD), lambda l: (0, 0)),
            pl.BlockSpec((1, HDD), lambda l: (0, 0)),
            pl.BlockSpec((HDD, HDD), lambda l: (0, 0)),
            pl.BlockSpec((1, HDD), lambda l: (0, 0)),
        ],
        out_specs=[
            pl.BlockSpec((_LB, BB, HDD), lambda l: (l, 0, 0)),
            pl.BlockSpec((1, 1), lambda l: (0, 0)),
        ],
        out_shape=[
            jax.ShapeDtypeStruct((LL, BB, HDD), jnp.float32),
            jax.ShapeDtypeStruct((1, 1), jnp.float32),
        ],
        scratch_shapes=[pltpu.SMEM((1,), jnp.float32)],
    )(d_seq, eps, ca.reshape(1, BB, HDD),
      p['m1W'], p['m1b'].reshape(1, HDD), p['m2W'], p['m2b'].reshape(1, HDD),
      p['v1W'], p['v1b'].reshape(1, HDD), p['v2W'], p['v2b'].reshape(1, HDD))


def _segmax_body(xb, ob):
    ob[...] = jnp.max(xb[...], axis=1, keepdims=True)


def _segmax(xr3):
    out = pl.pallas_call(
        _segmax_body,
        grid=(BB,),
        in_specs=[pl.BlockSpec((1, LL, HDP), lambda b: (b, 0, 0))],
        out_specs=pl.BlockSpec((1, 1, HDP), lambda b: (b, 0, 0)),
        out_shape=jax.ShapeDtypeStruct((BB, 1, HDP), jnp.float32),
    )(xr3)
    return out.reshape(BB, HDP)


def _pmvo_body(xb, w1, b1, w2, b2, ob):
    h = jnp.maximum(
        jnp.dot(xb[...], w1[...], preferred_element_type=jnp.float32) + b1[...], 0.0)
    ob[...] = jnp.dot(h, w2[...], preferred_element_type=jnp.float32) + b2[...]


def _pmvo(x2, f1wp, f1b, f2w, f2b):
    return pl.pallas_call(
        _pmvo_body,
        in_specs=[
            pl.BlockSpec((BB, HDP), lambda: (0, 0)),
            pl.BlockSpec((HDP, 1024), lambda: (0, 0)),
            pl.BlockSpec((1, 1024), lambda: (0, 0)),
            pl.BlockSpec((1024, 128), lambda: (0, 0)),
            pl.BlockSpec((1, 128), lambda: (0, 0)),
        ],
        out_specs=pl.BlockSpec((BB, 128), lambda: (0, 0)),
        out_shape=jax.ShapeDtypeStruct((BB, 128), jnp.float32),
    )(x2, f1wp, f1b.reshape(1, 1024), f2w, f2b.reshape(1, 128))


def _pad2(a, r, c):
    return jnp.pad(a, ((0, r - a.shape[0]), (0, c - a.shape[1])))


def kernel(x, edge_index, batch, num_nodes, y, con, eps, params):
    p = params
    src = edge_index[0]
    dst = edge_index[1]

    ones = jnp.ones((NN, FC), jnp.float32)
    degfull = _agg(1, src, dst, [ones])
    dinv = _dinv(degfull)

    def layer(xin, C, fin, W, b):
        wp = _pad2(W, fin, C * FC)
        bp = jnp.pad(b, (0, C * FC - b.shape[0]))
        h2 = _mm(C, fin, xin, wp, dinv)
        hchunks = [h2[:, c * FC:(c + 1) * FC] for c in range(C)]
        agg = _agg(C, src, dst, hchunks)
        aggchunks = [agg[c * NN:(c + 1) * NN] for c in range(C)]
        return _epi(C, aggchunks, dinv, bp)

    xp = jnp.pad(x, ((0, 0), (0, 2)))
    x1 = layer(xp, 2, 96, p['W1'], p['b1'])
    x2in = layer(x1, 3, 2 * FC, p['W2'], p['b2'])
    xr = layer(x2in, 3, 3 * FC, p['W3'], p['b3'])

    ca = _cond(con, p['condW'], p['condb'], y)
    col0 = xr[:, 0].reshape(BB, LL)
    d_seq, mask = _trans(xr[:, :HDD], col0, p['pp'])
    z, kl2 = _vae(d_seq, eps, ca, p)

    x2 = _segmax(xr.reshape(BB, LL, HDP))
    f1wp = jnp.pad(p['f1W'], ((0, HDP - HDD), (0, 0)))
    pmvo = _pmvo(x2, f1wp, p['f1b'], p['f2W'], p['f2b'])

    return d_seq, z, mask, pmvo, kl2[0, 0]
